# Initial kernel scaffold; baseline (speedup 1.0000x reference)
import jax
import jax.numpy as jnp
from jax.experimental import pallas as pl


def kernel(depth_ids, purpose_ids, depth_table, purpose_table, W1, b1, W2, b2):
    raise NotImplementedError("write your pallas kernel here")


# speedup vs baseline: 1.1525x; 1.1525x over previous
import functools

import jax
import jax.numpy as jnp
from jax import lax
from jax.experimental import pallas as pl
from jax.experimental.pallas import tpu as pltpu
from jax.experimental.pallas import tpu_sc as plsc

NC = 2
NS = 16
NW = NC * NS
CHUNK = 128

VOCAB = 100000
EMB = 64
BATCH = 16384

B_PER_W = BATCH // NW
CHUNKS_PER_W = B_PER_W // CHUNK


def _gather_body(d_table, p_table, d_ids, p_ids, d_out, p_out,
                 idx_d, idx_p, rows_d, rows_p, sem):
    wid = lax.axis_index("s") * NC + lax.axis_index("c")
    rbase = wid * CHUNKS_PER_W
    pltpu.sync_copy(d_ids.at[pl.ds(rbase, CHUNKS_PER_W)], idx_d)
    pltpu.sync_copy(p_ids.at[pl.ds(rbase, CHUNKS_PER_W)], idx_p)
    copies = []
    for j in range(CHUNKS_PER_W):
        copies.append(pltpu.async_copy(
            d_table.at[idx_d.at[j]], rows_d.at[pl.ds(j * CHUNK, CHUNK)], sem))
        copies.append(pltpu.async_copy(
            p_table.at[idx_p.at[j]], rows_p.at[pl.ds(j * CHUNK, CHUNK)], sem))
    for c in copies:
        c.wait()
    base = wid * B_PER_W
    pltpu.sync_copy(rows_d, d_out.at[pl.ds(base, B_PER_W)])
    pltpu.sync_copy(rows_p, p_out.at[pl.ds(base, B_PER_W)])


_gather = pl.kernel(
    _gather_body,
    out_type=(
        jax.ShapeDtypeStruct((BATCH, EMB), jnp.float32),
        jax.ShapeDtypeStruct((BATCH, EMB), jnp.float32),
    ),
    mesh=plsc.VectorSubcoreMesh(
        core_axis_name="c", subcore_axis_name="s",
        num_cores=NC, num_subcores=NS),
    scratch_types=[
        pltpu.VMEM((CHUNKS_PER_W, CHUNK), jnp.int32),
        pltpu.VMEM((CHUNKS_PER_W, CHUNK), jnp.int32),
        pltpu.VMEM((B_PER_W, EMB), jnp.float32),
        pltpu.VMEM((B_PER_W, EMB), jnp.float32),
        pltpu.SemaphoreType.DMA,
    ],
    compiler_params=pltpu.CompilerParams(use_tc_tiling_on_sc=False),
)


def _mlp_body(d_ref, p_ref, w1a_ref, w1b_ref, b1_ref, w2_ref, b2_ref, o_ref):
    h = jnp.dot(d_ref[...], w1a_ref[...], preferred_element_type=jnp.float32)
    h = h + jnp.dot(p_ref[...], w1b_ref[...], preferred_element_type=jnp.float32)
    h = jnp.maximum(h + b1_ref[...], 0.0)
    o = jnp.dot(h, w2_ref[...], preferred_element_type=jnp.float32) + b2_ref[...]
    o_ref[...] = 1.0 / (1.0 + jnp.exp(-o))


BB = 2048


def _mlp(d_emb, p_emb, w1a, w1b, b1, w2, b2):
    grid = (BATCH // BB,)
    return pl.pallas_call(
        _mlp_body,
        grid=grid,
        in_specs=[
            pl.BlockSpec((BB, EMB), lambda i: (i, 0)),
            pl.BlockSpec((BB, EMB), lambda i: (i, 0)),
            pl.BlockSpec((EMB, 32), lambda i: (0, 0)),
            pl.BlockSpec((EMB, 32), lambda i: (0, 0)),
            pl.BlockSpec((1, 32), lambda i: (0, 0)),
            pl.BlockSpec((32, 4), lambda i: (0, 0)),
            pl.BlockSpec((1, 4), lambda i: (0, 0)),
        ],
        out_specs=pl.BlockSpec((BB, 4), lambda i: (i, 0)),
        out_shape=jax.ShapeDtypeStruct((BATCH, 4), jnp.float32),
    )(d_emb, p_emb, w1a, w1b, b1, w2, b2)


@jax.jit
def kernel(depth_ids, purpose_ids, depth_table, purpose_table, W1, b1, W2, b2):
    d_ids = depth_ids.astype(jnp.int32).reshape(NW * CHUNKS_PER_W, CHUNK)
    p_ids = purpose_ids.astype(jnp.int32).reshape(NW * CHUNKS_PER_W, CHUNK)
    d_emb, p_emb = _gather(depth_table, purpose_table, d_ids, p_ids)
    w1a = W1[:EMB]
    w1b = W1[EMB:]
    return _mlp(d_emb, p_emb, w1a, w1b,
                b1.reshape(1, 32), W2, b2.reshape(1, 4))
